# baseline (speedup 1.0000x reference)
import functools

import jax
import jax.numpy as jnp
from jax import lax
from jax.experimental import pallas as pl
from jax.experimental.pallas import tpu as pltpu
from jax.experimental.pallas import tpu_sc as plsc

N = 2048
E = 65536
C = 256
H = 4
HD = C // H
MAX_HOPS = 8

BR = 256
NRB = N // BR
INF_B = 30000.0


SLAB = 512
EPS = E // 16
NCH = EPS // 128
RPS = SLAB // 16
WPS = RPS * N
ZW = 8192


def _edge_counts_body(ei, mc, ac, dst_v, src_v, idx_b, val_b, zbuf, iz, vz,
                      shared):
    c = lax.axis_index("c")
    s = lax.axis_index("s")

    def _z(i, _):
        zbuf[pl.ds(i * 16, 16)] = jnp.zeros((16,), jnp.float32)
        return _
    lax.fori_loop(0, ZW // 16, _z, None)
    for u in range(8):
        iz[0, pl.ds(u * 16, 16)] = jnp.zeros((16,), jnp.int32)
        vz[0, pl.ds(u * 16, 16)] = jnp.zeros((16,), jnp.float32)
    pltpu.sync_copy(ei.at[0, pl.ds(s * EPS, EPS)], src_v)
    pltpu.sync_copy(ei.at[1, pl.ds(s * EPS, EPS)], dst_v)

    for phase in range(4):
        is_m = phase < 2
        t = phase % 2
        base = (c * 2 + t) * SLAB

        for z in range(WPS // ZW):
            pltpu.sync_copy(zbuf, shared.at[pl.ds(s * WPS + z * ZW, ZW)])
        plsc.subcore_barrier()

        def _build(j, _):
            for u in range(8):
                o = j * 128 + u * 16
                d = dst_v[pl.ds(o, 16)]
                sv = src_v[pl.ds(o, 16)]
                row = (d if is_m else sv) - base
                col = sv if is_m else d
                ok = (row >= 0) & (row < SLAB)
                idx_b[j, pl.ds(u * 16, 16)] = jnp.where(ok, row * N + col, 0)
                val_b[j, pl.ds(u * 16, 16)] = jnp.where(ok, 1.0, 0.0)
            return _
        lax.fori_loop(0, NCH, _build, None)
        plsc.subcore_barrier()

        for j in range(NCH):
            pltpu.sync_copy(val_b.at[j], shared.at[idx_b.at[j]], add=True)
        for _ in range(2):
            pltpu.sync_copy(vz.at[0], shared.at[iz.at[0]], add=True)
        plsc.subcore_barrier()

        out = mc if is_m else ac
        pltpu.sync_copy(shared.at[pl.ds(s * WPS, WPS)],
                        out.at[pl.ds((c * 32 + t * 16 + s) * WPS, WPS)])
        plsc.subcore_barrier()


def _edge_counts(edge_index):
    out = jax.ShapeDtypeStruct((N * N,), jnp.float32)
    k = pl.kernel(
        _edge_counts_body,
        mesh=plsc.VectorSubcoreMesh(core_axis_name="c", subcore_axis_name="s"),
        out_type=(out, out),
        scratch_types=[
            pltpu.VMEM((EPS,), jnp.int32),
            pltpu.VMEM((EPS,), jnp.int32),
            pltpu.VMEM((NCH, 128), jnp.int32),
            pltpu.VMEM((NCH, 128), jnp.float32),
            pltpu.VMEM((ZW,), jnp.float32),
            pltpu.VMEM((1, 128), jnp.int32),
            pltpu.VMEM((1, 128), jnp.float32),
            pltpu.VMEM_SHARED((SLAB * N,), jnp.float32),
        ],
    )
    mc, ac = k(edge_index)
    return mc.reshape(N, N), ac.reshape(N, N)


def _qkv_body(x_ref, wq_ref, wk_ref, wv_ref, bq_ref, bk_ref, bv_ref,
              q_ref, k_ref, v_ref):
    x = x_ref[...]
    hp = jax.lax.Precision.HIGHEST
    q_ref[0] = jnp.dot(x, wq_ref[0], precision=hp) + bq_ref[0]
    k_ref[0] = jnp.dot(x, wk_ref[0], precision=hp) + bk_ref[0]
    v_ref[0] = jnp.dot(x, wv_ref[0], precision=hp) + bv_ref[0]


def _qkv(x, Wq, Wk, Wv, bq, bk, bv):
    out = jax.ShapeDtypeStruct((H, N, HD), jnp.float32)
    wspec = pl.BlockSpec((1, C, HD), lambda h: (h, 0, 0))
    bspec = pl.BlockSpec((1, 1, HD), lambda h: (h, 0, 0))
    ospec = pl.BlockSpec((1, N, HD), lambda h: (h, 0, 0))
    wh = lambda W: W.reshape(C, H, HD).transpose(1, 0, 2)
    bh = lambda b: b.reshape(1, H, HD).transpose(1, 0, 2)
    return pl.pallas_call(
        _qkv_body,
        grid=(H,),
        in_specs=[pl.BlockSpec((N, C), lambda h: (0, 0))] + [wspec] * 3
                 + [bspec] * 3,
        out_specs=(ospec, ospec, ospec),
        out_shape=(out, out, out),
    )(x, wh(Wq), wh(Wk), wh(Wv), bh(bq), bh(bk), bh(bv))


def _bfs_body(acnt_ref, out_ref, abf_scr, reach_scr, dist_scr):
    k = pl.program_id(0)
    r = pl.program_id(1)
    rows = pl.ds(r * BR, BR)

    @pl.when(k == 0)
    def _init():
        a01 = acnt_ref[...] > 0.0
        abf_scr[rows, :] = a01.astype(jnp.bfloat16)
        reach_scr[rows, :] = a01.astype(jnp.bfloat16)
        ii = jax.lax.broadcasted_iota(jnp.int32, (BR, N), 0) + r * BR
        jj = jax.lax.broadcasted_iota(jnp.int32, (BR, N), 1)
        d = jnp.where(a01, 1.0, INF_B)
        d = jnp.where(ii == jj, 0.0, d)
        dist_scr[rows, :] = d.astype(jnp.bfloat16)

    @pl.when(k > 0)
    def _step():
        cnt = jnp.dot(reach_scr[rows, :], abf_scr[...],
                      preferred_element_type=jnp.float32)
        new = cnt > 0.0
        d = dist_scr[rows, :]
        hop = (k + 1).astype(jnp.float32).astype(jnp.bfloat16)
        dist_scr[rows, :] = jnp.where(new & (d > 1000.0), hop, d)
        reach_scr[rows, :] = new.astype(jnp.bfloat16)

    @pl.when(k == MAX_HOPS - 1)
    def _emit():
        d = dist_scr[rows, :].astype(jnp.float32)
        out_ref[...] = jnp.where(d > 1000.0, -1.0, d)


def _bfs_bias(Acnt):
    return pl.pallas_call(
        _bfs_body,
        grid=(MAX_HOPS, NRB),
        in_specs=[pl.BlockSpec((BR, N),
                               lambda k, r: (jnp.where(k == 0, r, 0), 0))],
        out_specs=pl.BlockSpec(
            (BR, N), lambda k, r: (jnp.where(k == MAX_HOPS - 1, r, 0), 0)),
        out_shape=jax.ShapeDtypeStruct((N, N), jnp.float32),
        scratch_shapes=[
            pltpu.VMEM((N, N), jnp.bfloat16),
            pltpu.VMEM((N, N), jnp.bfloat16),
            pltpu.VMEM((N, N), jnp.bfloat16),
        ],
    )(Acnt)


def _attn_body(q_ref, k_ref, v_ref, bias_ref, mc_ref, ac_ref, x_ref,
               eb_ref, g_ref, b_ref, y_ref):
    hp = jax.lax.Precision.HIGHEST
    mcnt = mc_ref[...]
    mask = mcnt > 0.0
    base = bias_ref[...] + eb_ref[0, 0]
    outs = []
    for h in range(H):
        q = q_ref[h]
        s = jax.lax.dot_general(q, k_ref[h], (((1,), (1,)), ((), ())),
                                precision=hp) * (1.0 / (HD ** 0.5))
        s = s + base
        sm = jnp.where(mask, s, -1e30)
        m = jnp.max(sm, axis=1, keepdims=True)
        m = jnp.where(m < -1e29, 0.0, m)
        ex = mcnt * jnp.exp(sm - m)
        ssum = jnp.sum(ex, axis=1, keepdims=True)
        outs.append(jnp.dot(ex, v_ref[h], precision=hp) / (ssum + 1e-16))
    acc = jnp.concatenate(outs, axis=1)
    in_deg = jnp.sum(mcnt, axis=1, keepdims=True)
    out_deg = jnp.sum(ac_ref[...], axis=1, keepdims=True)
    hh = acc + x_ref[...] + (in_deg + out_deg)
    mu = jnp.mean(hh, axis=1, keepdims=True)
    var = jnp.mean((hh - mu) ** 2, axis=1, keepdims=True)
    y = (hh - mu) * jax.lax.rsqrt(var + 1e-5)
    y_ref[...] = y * g_ref[...] + b_ref[...]


def _attention(Q, K, V, bias, Mc, Ac, x, eb, gamma, beta):
    return pl.pallas_call(
        _attn_body,
        grid=(NRB,),
        in_specs=[
            pl.BlockSpec((H, BR, HD), lambda r: (0, r, 0)),
            pl.BlockSpec((H, N, HD), lambda r: (0, 0, 0)),
            pl.BlockSpec((H, N, HD), lambda r: (0, 0, 0)),
            pl.BlockSpec((BR, N), lambda r: (r, 0)),
            pl.BlockSpec((BR, N), lambda r: (r, 0)),
            pl.BlockSpec((BR, N), lambda r: (r, 0)),
            pl.BlockSpec((BR, C), lambda r: (r, 0)),
            pl.BlockSpec((1, 1), lambda r: (0, 0)),
            pl.BlockSpec((1, C), lambda r: (0, 0)),
            pl.BlockSpec((1, C), lambda r: (0, 0)),
        ],
        out_specs=pl.BlockSpec((BR, C), lambda r: (r, 0)),
        out_shape=jax.ShapeDtypeStruct((N, C), jnp.float32),
    )(Q, K, V, bias, Mc, Ac, x, eb.reshape(1, 1), gamma.reshape(1, C),
      beta.reshape(1, C))


def kernel(x, edge_index, Wq, bq, Wk, bk, Wv, bv, edge_bias, gamma, beta):
    Mc, Ac = _edge_counts(edge_index)
    Q, K, V = _qkv(x, Wq, Wk, Wv, bq, bk, bv)
    bias = _bfs_bias(Ac)
    return _attention(Q, K, V, bias, Mc, Ac, x, edge_bias, gamma, beta)


# speedup vs baseline: 1.0064x; 1.0064x over previous
import functools

import jax
import jax.numpy as jnp
from jax import lax
from jax.experimental import pallas as pl
from jax.experimental.pallas import tpu as pltpu
from jax.experimental.pallas import tpu_sc as plsc

N = 2048
E = 65536
C = 256
H = 4
HD = C // H
MAX_HOPS = 8

BR = 256
NRB = N // BR
INF_B = 30000.0


SLAB = 512
EPS = E // 16
NCH = EPS // 128
RPS = SLAB // 16
WPS = RPS * N
ZW = 8192


def _edge_counts_body(ei, mc, ac, dst_v, src_v, idx_b, val_b, zbuf, iz, vz,
                      shared, sem):
    c = lax.axis_index("c")
    s = lax.axis_index("s")

    cp_src = pltpu.async_copy(ei.at[0, pl.ds(s * EPS, EPS)], src_v, sem)
    cp_dst = pltpu.async_copy(ei.at[1, pl.ds(s * EPS, EPS)], dst_v, sem)

    def _z(i, _):
        zbuf[pl.ds(i * 16, 16)] = jnp.zeros((16,), jnp.float32)
        return _
    lax.fori_loop(0, ZW // 16, _z, None)
    for u in range(8):
        iz[0, pl.ds(u * 16, 16)] = jnp.zeros((16,), jnp.int32)
        vz[0, pl.ds(u * 16, 16)] = jnp.zeros((16,), jnp.float32)
    cp_src.wait()
    cp_dst.wait()

    for phase in range(4):
        is_m = phase < 2
        t = phase % 2
        base = (c * 2 + t) * SLAB

        zcps = [pltpu.async_copy(
                    zbuf, shared.at[pl.ds(s * WPS + z * ZW, ZW)], sem)
                for z in range(WPS // ZW)]

        def _build(j, _):
            for u in range(8):
                o = j * 128 + u * 16
                d = dst_v[pl.ds(o, 16)]
                sv = src_v[pl.ds(o, 16)]
                row = (d if is_m else sv) - base
                col = sv if is_m else d
                ok = (row >= 0) & (row < SLAB)
                idx_b[j, pl.ds(u * 16, 16)] = jnp.where(ok, row * N + col, 0)
                val_b[j, pl.ds(u * 16, 16)] = jnp.where(ok, 1.0, 0.0)
            return _
        lax.fori_loop(0, NCH, _build, None)
        for cp in zcps:
            cp.wait()
        plsc.subcore_barrier()

        cps = [pltpu.async_copy(val_b.at[j], shared.at[idx_b.at[j]], sem,
                                add=True)
               for j in range(NCH)]
        for cp in cps:
            cp.wait()
        for _ in range(2):
            pltpu.sync_copy(vz.at[0], shared.at[iz.at[0]], add=True)
        plsc.subcore_barrier()

        out = mc if is_m else ac
        pltpu.sync_copy(shared.at[pl.ds(s * WPS, WPS)],
                        out.at[pl.ds((c * 32 + t * 16 + s) * WPS, WPS)])
        plsc.subcore_barrier()


def _edge_counts(edge_index):
    out = jax.ShapeDtypeStruct((N * N,), jnp.float32)
    k = pl.kernel(
        _edge_counts_body,
        mesh=plsc.VectorSubcoreMesh(core_axis_name="c", subcore_axis_name="s"),
        out_type=(out, out),
        scratch_types=[
            pltpu.VMEM((EPS,), jnp.int32),
            pltpu.VMEM((EPS,), jnp.int32),
            pltpu.VMEM((NCH, 128), jnp.int32),
            pltpu.VMEM((NCH, 128), jnp.float32),
            pltpu.VMEM((ZW,), jnp.float32),
            pltpu.VMEM((1, 128), jnp.int32),
            pltpu.VMEM((1, 128), jnp.float32),
            pltpu.VMEM_SHARED((SLAB * N,), jnp.float32),
            pltpu.SemaphoreType.DMA,
        ],
    )
    mc, ac = k(edge_index)
    return mc.reshape(N, N), ac.reshape(N, N)


def _qkv_body(x_ref, wq_ref, wk_ref, wv_ref, bq_ref, bk_ref, bv_ref,
              q_ref, k_ref, v_ref):
    x = x_ref[...]
    hp = jax.lax.Precision.HIGHEST
    q_ref[0] = jnp.dot(x, wq_ref[0], precision=hp) + bq_ref[0]
    k_ref[0] = jnp.dot(x, wk_ref[0], precision=hp) + bk_ref[0]
    v_ref[0] = jnp.dot(x, wv_ref[0], precision=hp) + bv_ref[0]


def _qkv(x, Wq, Wk, Wv, bq, bk, bv):
    out = jax.ShapeDtypeStruct((H, N, HD), jnp.float32)
    wspec = pl.BlockSpec((1, C, HD), lambda h: (h, 0, 0))
    bspec = pl.BlockSpec((1, 1, HD), lambda h: (h, 0, 0))
    ospec = pl.BlockSpec((1, N, HD), lambda h: (h, 0, 0))
    wh = lambda W: W.reshape(C, H, HD).transpose(1, 0, 2)
    bh = lambda b: b.reshape(1, H, HD).transpose(1, 0, 2)
    return pl.pallas_call(
        _qkv_body,
        grid=(H,),
        in_specs=[pl.BlockSpec((N, C), lambda h: (0, 0))] + [wspec] * 3
                 + [bspec] * 3,
        out_specs=(ospec, ospec, ospec),
        out_shape=(out, out, out),
    )(x, wh(Wq), wh(Wk), wh(Wv), bh(bq), bh(bk), bh(bv))


def _bfs_body(acnt_ref, out_ref, abf_scr, reach_scr, dist_scr):
    k = pl.program_id(0)
    r = pl.program_id(1)
    rows = pl.ds(r * BR, BR)

    @pl.when(k == 0)
    def _init():
        a01 = acnt_ref[...] > 0.0
        abf_scr[rows, :] = a01.astype(jnp.bfloat16)
        reach_scr[rows, :] = a01.astype(jnp.bfloat16)
        ii = jax.lax.broadcasted_iota(jnp.int32, (BR, N), 0) + r * BR
        jj = jax.lax.broadcasted_iota(jnp.int32, (BR, N), 1)
        d = jnp.where(a01, 1.0, INF_B)
        d = jnp.where(ii == jj, 0.0, d)
        dist_scr[rows, :] = d.astype(jnp.bfloat16)

    @pl.when(k > 0)
    def _step():
        cnt = jnp.dot(reach_scr[rows, :], abf_scr[...],
                      preferred_element_type=jnp.float32)
        new = cnt > 0.0
        d = dist_scr[rows, :]
        hop = (k + 1).astype(jnp.float32).astype(jnp.bfloat16)
        dist_scr[rows, :] = jnp.where(new & (d > 1000.0), hop, d)
        reach_scr[rows, :] = new.astype(jnp.bfloat16)

    @pl.when(k == MAX_HOPS - 1)
    def _emit():
        d = dist_scr[rows, :].astype(jnp.float32)
        out_ref[...] = jnp.where(d > 1000.0, -1.0, d)


def _bfs_bias(Acnt):
    return pl.pallas_call(
        _bfs_body,
        grid=(MAX_HOPS, NRB),
        in_specs=[pl.BlockSpec((BR, N),
                               lambda k, r: (jnp.where(k == 0, r, 0), 0))],
        out_specs=pl.BlockSpec(
            (BR, N), lambda k, r: (jnp.where(k == MAX_HOPS - 1, r, 0), 0)),
        out_shape=jax.ShapeDtypeStruct((N, N), jnp.float32),
        scratch_shapes=[
            pltpu.VMEM((N, N), jnp.bfloat16),
            pltpu.VMEM((N, N), jnp.bfloat16),
            pltpu.VMEM((N, N), jnp.bfloat16),
        ],
    )(Acnt)


def _attn_body(q_ref, k_ref, v_ref, bias_ref, mc_ref, ac_ref, x_ref,
               eb_ref, g_ref, b_ref, y_ref):
    hp = jax.lax.Precision.HIGHEST
    mcnt = mc_ref[...]
    mask = mcnt > 0.0
    base = bias_ref[...] + eb_ref[0, 0]
    outs = []
    for h in range(H):
        q = q_ref[h]
        s = jax.lax.dot_general(q, k_ref[h], (((1,), (1,)), ((), ())),
                                precision=hp) * (1.0 / (HD ** 0.5))
        s = s + base
        sm = jnp.where(mask, s, -1e30)
        m = jnp.max(sm, axis=1, keepdims=True)
        m = jnp.where(m < -1e29, 0.0, m)
        ex = mcnt * jnp.exp(sm - m)
        ssum = jnp.sum(ex, axis=1, keepdims=True)
        outs.append(jnp.dot(ex, v_ref[h], precision=hp) / (ssum + 1e-16))
    acc = jnp.concatenate(outs, axis=1)
    in_deg = jnp.sum(mcnt, axis=1, keepdims=True)
    out_deg = jnp.sum(ac_ref[...], axis=1, keepdims=True)
    hh = acc + x_ref[...] + (in_deg + out_deg)
    mu = jnp.mean(hh, axis=1, keepdims=True)
    var = jnp.mean((hh - mu) ** 2, axis=1, keepdims=True)
    y = (hh - mu) * jax.lax.rsqrt(var + 1e-5)
    y_ref[...] = y * g_ref[...] + b_ref[...]


def _attention(Q, K, V, bias, Mc, Ac, x, eb, gamma, beta):
    return pl.pallas_call(
        _attn_body,
        grid=(NRB,),
        in_specs=[
            pl.BlockSpec((H, BR, HD), lambda r: (0, r, 0)),
            pl.BlockSpec((H, N, HD), lambda r: (0, 0, 0)),
            pl.BlockSpec((H, N, HD), lambda r: (0, 0, 0)),
            pl.BlockSpec((BR, N), lambda r: (r, 0)),
            pl.BlockSpec((BR, N), lambda r: (r, 0)),
            pl.BlockSpec((BR, N), lambda r: (r, 0)),
            pl.BlockSpec((BR, C), lambda r: (r, 0)),
            pl.BlockSpec((1, 1), lambda r: (0, 0)),
            pl.BlockSpec((1, C), lambda r: (0, 0)),
            pl.BlockSpec((1, C), lambda r: (0, 0)),
        ],
        out_specs=pl.BlockSpec((BR, C), lambda r: (r, 0)),
        out_shape=jax.ShapeDtypeStruct((N, C), jnp.float32),
    )(Q, K, V, bias, Mc, Ac, x, eb.reshape(1, 1), gamma.reshape(1, C),
      beta.reshape(1, C))


def kernel(x, edge_index, Wq, bq, Wk, bk, Wv, bv, edge_bias, gamma, beta):
    Mc, Ac = _edge_counts(edge_index)
    Q, K, V = _qkv(x, Wq, Wk, Wv, bq, bk, bv)
    bias = _bfs_bias(Ac)
    return _attention(Q, K, V, bias, Mc, Ac, x, edge_bias, gamma, beta)


# speedup vs baseline: 1.2809x; 1.2727x over previous
import functools

import jax
import jax.numpy as jnp
from jax import lax
from jax.experimental import pallas as pl
from jax.experimental.pallas import tpu as pltpu
from jax.experimental.pallas import tpu_sc as plsc

N = 2048
E = 65536
C = 256
H = 4
HD = C // H
MAX_HOPS = 8

BR = 256
NRB = N // BR
INF_B = 30000.0


SLAB = 512
EPS = E // 16
NCH = EPS // 128
RPS = SLAB // 16
WPS = RPS * N
ZW = 8192


def _count_body(row_is_dst, ei, out, dst_v, src_v, idx_b, val_b, zbuf, iz, vz,
                shared, sem):
    c = lax.axis_index("c")
    s = lax.axis_index("s")

    cp_src = pltpu.async_copy(ei.at[0, pl.ds(s * EPS, EPS)], src_v, sem)
    cp_dst = pltpu.async_copy(ei.at[1, pl.ds(s * EPS, EPS)], dst_v, sem)

    def _z(i, _):
        zbuf[pl.ds(i * 16, 16)] = jnp.zeros((16,), jnp.float32)
        return _
    lax.fori_loop(0, ZW // 16, _z, None)
    for u in range(8):
        iz[0, pl.ds(u * 16, 16)] = jnp.zeros((16,), jnp.int32)
        vz[0, pl.ds(u * 16, 16)] = jnp.zeros((16,), jnp.float32)
    cp_src.wait()
    cp_dst.wait()

    for t in range(2):
        base = (c * 2 + t) * SLAB

        zcps = [pltpu.async_copy(
                    zbuf, shared.at[pl.ds(s * WPS + z * ZW, ZW)], sem)
                for z in range(WPS // ZW)]

        def _build(j, _):
            for u in range(8):
                o = j * 128 + u * 16
                d = dst_v[pl.ds(o, 16)]
                sv = src_v[pl.ds(o, 16)]
                row = (d if row_is_dst else sv) - base
                col = sv if row_is_dst else d
                ok = (row >= 0) & (row < SLAB)
                idx_b[j, pl.ds(u * 16, 16)] = jnp.where(ok, row * N + col, 0)
                val_b[j, pl.ds(u * 16, 16)] = jnp.where(ok, 1.0, 0.0)
            return _
        lax.fori_loop(0, NCH, _build, None)
        for cp in zcps:
            cp.wait()
        plsc.subcore_barrier()

        cps = [pltpu.async_copy(val_b.at[j], shared.at[idx_b.at[j]], sem,
                                add=True)
               for j in range(NCH)]
        for cp in cps:
            cp.wait()
        for _ in range(2):
            pltpu.sync_copy(vz.at[0], shared.at[iz.at[0]], add=True)
        plsc.subcore_barrier()

        pltpu.sync_copy(shared.at[pl.ds(s * WPS, WPS)],
                        out.at[pl.ds((c * 32 + t * 16 + s) * WPS, WPS)])
        plsc.subcore_barrier()


def _count_matrix(edge_index, row_is_dst):
    k = pl.kernel(
        functools.partial(_count_body, row_is_dst),
        mesh=plsc.VectorSubcoreMesh(core_axis_name="c", subcore_axis_name="s"),
        out_type=jax.ShapeDtypeStruct((N * N,), jnp.float32),
        scratch_types=[
            pltpu.VMEM((EPS,), jnp.int32),
            pltpu.VMEM((EPS,), jnp.int32),
            pltpu.VMEM((NCH, 128), jnp.int32),
            pltpu.VMEM((NCH, 128), jnp.float32),
            pltpu.VMEM((ZW,), jnp.float32),
            pltpu.VMEM((1, 128), jnp.int32),
            pltpu.VMEM((1, 128), jnp.float32),
            pltpu.VMEM_SHARED((SLAB * N,), jnp.float32),
            pltpu.SemaphoreType.DMA,
        ],
    )
    return k(edge_index).reshape(N, N)


def _edge_counts(edge_index):
    return (_count_matrix(edge_index, True), _count_matrix(edge_index, False))


def _qkv_body(x_ref, wq_ref, wk_ref, wv_ref, bq_ref, bk_ref, bv_ref,
              q_ref, k_ref, v_ref):
    x = x_ref[...]
    hp = jax.lax.Precision.HIGHEST
    q_ref[0] = jnp.dot(x, wq_ref[0], precision=hp) + bq_ref[0]
    k_ref[0] = jnp.dot(x, wk_ref[0], precision=hp) + bk_ref[0]
    v_ref[0] = jnp.dot(x, wv_ref[0], precision=hp) + bv_ref[0]


def _qkv(x, Wq, Wk, Wv, bq, bk, bv):
    out = jax.ShapeDtypeStruct((H, N, HD), jnp.float32)
    wspec = pl.BlockSpec((1, C, HD), lambda h: (h, 0, 0))
    bspec = pl.BlockSpec((1, 1, HD), lambda h: (h, 0, 0))
    ospec = pl.BlockSpec((1, N, HD), lambda h: (h, 0, 0))
    wh = lambda W: W.reshape(C, H, HD).transpose(1, 0, 2)
    bh = lambda b: b.reshape(1, H, HD).transpose(1, 0, 2)
    return pl.pallas_call(
        _qkv_body,
        grid=(H,),
        in_specs=[pl.BlockSpec((N, C), lambda h: (0, 0))] + [wspec] * 3
                 + [bspec] * 3,
        out_specs=(ospec, ospec, ospec),
        out_shape=(out, out, out),
    )(x, wh(Wq), wh(Wk), wh(Wv), bh(bq), bh(bk), bh(bv))


def _bfs_body(acnt_ref, out_ref, abf_scr, reach_scr, dist_scr):
    k = pl.program_id(0)
    r = pl.program_id(1)
    rows = pl.ds(r * BR, BR)

    @pl.when(k == 0)
    def _init():
        a01 = acnt_ref[...] > 0.0
        abf_scr[rows, :] = a01.astype(jnp.bfloat16)
        reach_scr[rows, :] = a01.astype(jnp.bfloat16)
        ii = jax.lax.broadcasted_iota(jnp.int32, (BR, N), 0) + r * BR
        jj = jax.lax.broadcasted_iota(jnp.int32, (BR, N), 1)
        d = jnp.where(a01, 1.0, INF_B)
        d = jnp.where(ii == jj, 0.0, d)
        dist_scr[rows, :] = d.astype(jnp.bfloat16)

    @pl.when(k > 0)
    def _step():
        cnt = jnp.dot(reach_scr[rows, :], abf_scr[...],
                      preferred_element_type=jnp.float32)
        new = cnt > 0.0
        d = dist_scr[rows, :]
        hop = (k + 1).astype(jnp.float32).astype(jnp.bfloat16)
        dist_scr[rows, :] = jnp.where(new & (d > 1000.0), hop, d)
        reach_scr[rows, :] = new.astype(jnp.bfloat16)

    @pl.when(k == MAX_HOPS - 1)
    def _emit():
        d = dist_scr[rows, :].astype(jnp.float32)
        out_ref[...] = jnp.where(d > 1000.0, -1.0, d)


def _bfs_bias(Acnt):
    return pl.pallas_call(
        _bfs_body,
        grid=(MAX_HOPS, NRB),
        in_specs=[pl.BlockSpec((BR, N),
                               lambda k, r: (jnp.where(k == 0, r, 0), 0))],
        out_specs=pl.BlockSpec(
            (BR, N), lambda k, r: (jnp.where(k == MAX_HOPS - 1, r, 0), 0)),
        out_shape=jax.ShapeDtypeStruct((N, N), jnp.float32),
        scratch_shapes=[
            pltpu.VMEM((N, N), jnp.bfloat16),
            pltpu.VMEM((N, N), jnp.bfloat16),
            pltpu.VMEM((N, N), jnp.bfloat16),
        ],
    )(Acnt)


def _attn_body(q_ref, k_ref, v_ref, bias_ref, mc_ref, ac_ref, x_ref,
               eb_ref, g_ref, b_ref, y_ref):
    hp = jax.lax.Precision.HIGHEST
    mcnt = mc_ref[...]
    mask = mcnt > 0.0
    base = bias_ref[...] + eb_ref[0, 0]
    outs = []
    for h in range(H):
        q = q_ref[h]
        s = jax.lax.dot_general(q, k_ref[h], (((1,), (1,)), ((), ())),
                                precision=hp) * (1.0 / (HD ** 0.5))
        s = s + base
        sm = jnp.where(mask, s, -1e30)
        m = jnp.max(sm, axis=1, keepdims=True)
        m = jnp.where(m < -1e29, 0.0, m)
        ex = mcnt * jnp.exp(sm - m)
        ssum = jnp.sum(ex, axis=1, keepdims=True)
        outs.append(jnp.dot(ex, v_ref[h], precision=hp) / (ssum + 1e-16))
    acc = jnp.concatenate(outs, axis=1)
    in_deg = jnp.sum(mcnt, axis=1, keepdims=True)
    out_deg = jnp.sum(ac_ref[...], axis=1, keepdims=True)
    hh = acc + x_ref[...] + (in_deg + out_deg)
    mu = jnp.mean(hh, axis=1, keepdims=True)
    var = jnp.mean((hh - mu) ** 2, axis=1, keepdims=True)
    y = (hh - mu) * jax.lax.rsqrt(var + 1e-5)
    y_ref[...] = y * g_ref[...] + b_ref[...]


def _attention(Q, K, V, bias, Mc, Ac, x, eb, gamma, beta):
    return pl.pallas_call(
        _attn_body,
        grid=(NRB,),
        in_specs=[
            pl.BlockSpec((H, BR, HD), lambda r: (0, r, 0)),
            pl.BlockSpec((H, N, HD), lambda r: (0, 0, 0)),
            pl.BlockSpec((H, N, HD), lambda r: (0, 0, 0)),
            pl.BlockSpec((BR, N), lambda r: (r, 0)),
            pl.BlockSpec((BR, N), lambda r: (r, 0)),
            pl.BlockSpec((BR, N), lambda r: (r, 0)),
            pl.BlockSpec((BR, C), lambda r: (r, 0)),
            pl.BlockSpec((1, 1), lambda r: (0, 0)),
            pl.BlockSpec((1, C), lambda r: (0, 0)),
            pl.BlockSpec((1, C), lambda r: (0, 0)),
        ],
        out_specs=pl.BlockSpec((BR, C), lambda r: (r, 0)),
        out_shape=jax.ShapeDtypeStruct((N, C), jnp.float32),
    )(Q, K, V, bias, Mc, Ac, x, eb.reshape(1, 1), gamma.reshape(1, C),
      beta.reshape(1, C))


def kernel(x, edge_index, Wq, bq, Wk, bk, Wv, bv, edge_bias, gamma, beta):
    Ac = _count_matrix(edge_index, False)
    Q, K, V = _qkv(x, Wq, Wk, Wv, bq, bk, bv)
    bias = _bfs_bias(Ac)
    Mc = _count_matrix(edge_index, True)
    return _attention(Q, K, V, bias, Mc, Ac, x, edge_bias, gamma, beta)


# speedup vs baseline: 1.3560x; 1.0586x over previous
import functools

import jax
import jax.numpy as jnp
from jax import lax
from jax.experimental import pallas as pl
from jax.experimental.pallas import tpu as pltpu
from jax.experimental.pallas import tpu_sc as plsc

N = 2048
E = 65536
C = 256
H = 4
HD = C // H
MAX_HOPS = 8

BR = 256
NRB = N // BR
INF_B = 30000.0


SLAB = 512
EPS = E // 16
NCH = EPS // 128
RPS = SLAB // 16
WPS = RPS * N
ZW = 8192


def _count_body(row_is_dst, ei, out, dst_v, src_v, idx_b, val_b, zbuf, iz, vz,
                shared, sem):
    c = lax.axis_index("c")
    s = lax.axis_index("s")

    cp_src = pltpu.async_copy(ei.at[0, pl.ds(s * EPS, EPS)], src_v, sem)
    cp_dst = pltpu.async_copy(ei.at[1, pl.ds(s * EPS, EPS)], dst_v, sem)

    def _z(i, _):
        zbuf[pl.ds(i * 16, 16)] = jnp.zeros((16,), jnp.float32)
        return _
    lax.fori_loop(0, ZW // 16, _z, None)
    for u in range(8):
        iz[0, pl.ds(u * 16, 16)] = jnp.zeros((16,), jnp.int32)
        vz[0, pl.ds(u * 16, 16)] = jnp.zeros((16,), jnp.float32)
    cp_src.wait()
    cp_dst.wait()

    for t in range(2):
        base = (c * 2 + t) * SLAB

        zcps = [pltpu.async_copy(
                    zbuf, shared.at[pl.ds(s * WPS + z * ZW, ZW)], sem)
                for z in range(WPS // ZW)]

        def _build(j, _):
            for u in range(8):
                o = j * 128 + u * 16
                d = dst_v[pl.ds(o, 16)]
                sv = src_v[pl.ds(o, 16)]
                row = (d if row_is_dst else sv) - base
                col = sv if row_is_dst else d
                ok = (row >= 0) & (row < SLAB)
                idx_b[j, pl.ds(u * 16, 16)] = jnp.where(ok, row * N + col, 0)
                val_b[j, pl.ds(u * 16, 16)] = jnp.where(ok, 1.0, 0.0)
            return _
        lax.fori_loop(0, NCH, _build, None)
        for cp in zcps:
            cp.wait()
        plsc.subcore_barrier()

        cps = [pltpu.async_copy(val_b.at[j], shared.at[idx_b.at[j]], sem,
                                add=True)
               for j in range(NCH)]
        for cp in cps:
            cp.wait()
        for _ in range(2):
            pltpu.sync_copy(vz.at[0], shared.at[iz.at[0]], add=True)
        plsc.subcore_barrier()

        pltpu.sync_copy(shared.at[pl.ds(s * WPS, WPS)],
                        out.at[pl.ds((c * 32 + t * 16 + s) * WPS, WPS)])
        plsc.subcore_barrier()


def _count_matrix(edge_index, row_is_dst):
    k = pl.kernel(
        functools.partial(_count_body, row_is_dst),
        mesh=plsc.VectorSubcoreMesh(core_axis_name="c", subcore_axis_name="s"),
        out_type=jax.ShapeDtypeStruct((N * N,), jnp.float32),
        scratch_types=[
            pltpu.VMEM((EPS,), jnp.int32),
            pltpu.VMEM((EPS,), jnp.int32),
            pltpu.VMEM((NCH, 128), jnp.int32),
            pltpu.VMEM((NCH, 128), jnp.float32),
            pltpu.VMEM((ZW,), jnp.float32),
            pltpu.VMEM((1, 128), jnp.int32),
            pltpu.VMEM((1, 128), jnp.float32),
            pltpu.VMEM_SHARED((SLAB * N,), jnp.float32),
            pltpu.SemaphoreType.DMA,
        ],
    )
    return k(edge_index).reshape(N, N)


def _edge_counts(edge_index):
    return (_count_matrix(edge_index, True), _count_matrix(edge_index, False))


def _qkv_body(x_ref, wq_ref, wk_ref, wv_ref, bq_ref, bk_ref, bv_ref,
              q_ref, k_ref, v_ref):
    x = x_ref[...]
    hp = jax.lax.Precision.HIGHEST
    q_ref[0] = jnp.dot(x, wq_ref[0], precision=hp) + bq_ref[0]
    k_ref[0] = jnp.dot(x, wk_ref[0], precision=hp) + bk_ref[0]
    v_ref[0] = jnp.dot(x, wv_ref[0], precision=hp) + bv_ref[0]


def _qkv(x, Wq, Wk, Wv, bq, bk, bv):
    out = jax.ShapeDtypeStruct((H, N, HD), jnp.float32)
    wspec = pl.BlockSpec((1, C, HD), lambda h: (h, 0, 0))
    bspec = pl.BlockSpec((1, 1, HD), lambda h: (h, 0, 0))
    ospec = pl.BlockSpec((1, N, HD), lambda h: (h, 0, 0))
    wh = lambda W: W.reshape(C, H, HD).transpose(1, 0, 2)
    bh = lambda b: b.reshape(1, H, HD).transpose(1, 0, 2)
    return pl.pallas_call(
        _qkv_body,
        grid=(H,),
        in_specs=[pl.BlockSpec((N, C), lambda h: (0, 0))] + [wspec] * 3
                 + [bspec] * 3,
        out_specs=(ospec, ospec, ospec),
        out_shape=(out, out, out),
    )(x, wh(Wq), wh(Wk), wh(Wv), bh(bq), bh(bk), bh(bv))


def _bfs_body(acnt_ref, out_ref, abf_scr, reach_scr, dist_scr, newc_s,
              done_s):
    k = pl.program_id(0)
    r = pl.program_id(1)
    rows = pl.ds(r * BR, BR)

    @pl.when((k == 0) & (r == 0))
    def _flags():
        newc_s[0] = 1
        done_s[0] = 0

    @pl.when((k > 0) & (r == 0))
    def _check():
        done_s[0] = jnp.where(newc_s[0] == 0, 1, done_s[0])
        newc_s[0] = 0

    @pl.when(k == 0)
    def _init():
        a01 = acnt_ref[...] > 0.0
        abf_scr[rows, :] = a01.astype(jnp.bfloat16)
        reach_scr[rows, :] = a01.astype(jnp.bfloat16)
        ii = jax.lax.broadcasted_iota(jnp.int32, (BR, N), 0) + r * BR
        jj = jax.lax.broadcasted_iota(jnp.int32, (BR, N), 1)
        d = jnp.where(a01, 1.0, INF_B)
        d = jnp.where(ii == jj, 0.0, d)
        dist_scr[rows, :] = d.astype(jnp.bfloat16)

    @pl.when((k > 0) & (done_s[0] == 0))
    def _step():
        cnt = jnp.dot(reach_scr[rows, :], abf_scr[...],
                      preferred_element_type=jnp.float32)
        new = cnt > 0.0
        d = dist_scr[rows, :]
        hop = (k + 1).astype(jnp.float32).astype(jnp.bfloat16)
        newly = new & (d > 1000.0)
        dist_scr[rows, :] = jnp.where(newly, hop, d)
        reach_scr[rows, :] = new.astype(jnp.bfloat16)
        newc_s[0] += jnp.sum(newly.astype(jnp.int32))

    @pl.when(k == MAX_HOPS - 1)
    def _emit():
        d = dist_scr[rows, :].astype(jnp.float32)
        out_ref[...] = jnp.where(d > 1000.0, -1.0, d)


def _bfs_bias(Acnt):
    return pl.pallas_call(
        _bfs_body,
        grid=(MAX_HOPS, NRB),
        in_specs=[pl.BlockSpec((BR, N),
                               lambda k, r: (jnp.where(k == 0, r, 0), 0))],
        out_specs=pl.BlockSpec(
            (BR, N), lambda k, r: (jnp.where(k == MAX_HOPS - 1, r, 0), 0)),
        out_shape=jax.ShapeDtypeStruct((N, N), jnp.float32),
        scratch_shapes=[
            pltpu.VMEM((N, N), jnp.bfloat16),
            pltpu.VMEM((N, N), jnp.bfloat16),
            pltpu.VMEM((N, N), jnp.bfloat16),
            pltpu.SMEM((1,), jnp.int32),
            pltpu.SMEM((1,), jnp.int32),
        ],
    )(Acnt)


def _attn_body(q_ref, k_ref, v_ref, bias_ref, mc_ref, ac_ref, x_ref,
               eb_ref, g_ref, b_ref, y_ref):
    hp = jax.lax.Precision.HIGHEST
    mcnt = mc_ref[...]
    mask = mcnt > 0.0
    base = bias_ref[...] + eb_ref[0, 0]
    outs = []
    for h in range(H):
        q = q_ref[h]
        s = jax.lax.dot_general(q, k_ref[h], (((1,), (1,)), ((), ())),
                                precision=hp) * (1.0 / (HD ** 0.5))
        s = s + base
        sm = jnp.where(mask, s, -1e30)
        m = jnp.max(sm, axis=1, keepdims=True)
        m = jnp.where(m < -1e29, 0.0, m)
        ex = mcnt * jnp.exp(sm - m)
        ssum = jnp.sum(ex, axis=1, keepdims=True)
        outs.append(jnp.dot(ex, v_ref[h], precision=hp) / (ssum + 1e-16))
    acc = jnp.concatenate(outs, axis=1)
    in_deg = jnp.sum(mcnt, axis=1, keepdims=True)
    out_deg = jnp.sum(ac_ref[...], axis=1, keepdims=True)
    hh = acc + x_ref[...] + (in_deg + out_deg)
    mu = jnp.mean(hh, axis=1, keepdims=True)
    var = jnp.mean((hh - mu) ** 2, axis=1, keepdims=True)
    y = (hh - mu) * jax.lax.rsqrt(var + 1e-5)
    y_ref[...] = y * g_ref[...] + b_ref[...]


def _attention(Q, K, V, bias, Mc, Ac, x, eb, gamma, beta):
    return pl.pallas_call(
        _attn_body,
        grid=(NRB,),
        in_specs=[
            pl.BlockSpec((H, BR, HD), lambda r: (0, r, 0)),
            pl.BlockSpec((H, N, HD), lambda r: (0, 0, 0)),
            pl.BlockSpec((H, N, HD), lambda r: (0, 0, 0)),
            pl.BlockSpec((BR, N), lambda r: (r, 0)),
            pl.BlockSpec((BR, N), lambda r: (r, 0)),
            pl.BlockSpec((BR, N), lambda r: (r, 0)),
            pl.BlockSpec((BR, C), lambda r: (r, 0)),
            pl.BlockSpec((1, 1), lambda r: (0, 0)),
            pl.BlockSpec((1, C), lambda r: (0, 0)),
            pl.BlockSpec((1, C), lambda r: (0, 0)),
        ],
        out_specs=pl.BlockSpec((BR, C), lambda r: (r, 0)),
        out_shape=jax.ShapeDtypeStruct((N, C), jnp.float32),
    )(Q, K, V, bias, Mc, Ac, x, eb.reshape(1, 1), gamma.reshape(1, C),
      beta.reshape(1, C))


def kernel(x, edge_index, Wq, bq, Wk, bk, Wv, bv, edge_bias, gamma, beta):
    Ac = _count_matrix(edge_index, False)
    Q, K, V = _qkv(x, Wq, Wk, Wv, bq, bk, bv)
    bias = _bfs_bias(Ac)
    Mc = _count_matrix(edge_index, True)
    return _attention(Q, K, V, bias, Mc, Ac, x, edge_bias, gamma, beta)
